# baseline (speedup 1.0000x reference)
import functools

import jax
import jax.numpy as jnp
from jax import lax
from jax.experimental import pallas as pl
from jax.experimental.pallas import tpu as pltpu
from jax.experimental.pallas import tpu_sc as plsc

NUM_EMB = 1000000
D = 64
B = 4096
S = 200
TOTAL = B * S
NC = 2
NS = 16
NW = NC * NS
PER_W = TOTAL // NW
CHUNK = 128
N_CHUNK = PER_W // CHUNK

_mesh = plsc.VectorSubcoreMesh(core_axis_name="c", subcore_axis_name="s")


@functools.partial(
    pl.kernel,
    out_type=jax.ShapeDtypeStruct((TOTAL, D), jnp.float32),
    mesh=_mesh,
    compiler_params=pltpu.CompilerParams(use_tc_tiling_on_sc=False),
    scratch_types=[
        pltpu.VMEM((N_CHUNK, CHUNK), jnp.int32),
        pltpu.VMEM((CHUNK, D), jnp.float32),
        pltpu.SemaphoreType.DMA,
    ],
)
def _embed_sc(table_hbm, idx_hbm, out_hbm, idx_v, rows_v, gsem):
    wid = lax.axis_index("s") * NC + lax.axis_index("c")
    pltpu.sync_copy(idx_hbm.at[pl.ds(wid * N_CHUNK, N_CHUNK)], idx_v)
    base = wid * PER_W

    @pl.loop(0, N_CHUNK)
    def _chunk(g):
        pltpu.async_copy(table_hbm.at[idx_v.at[g]], rows_v, gsem).wait()
        pltpu.sync_copy(rows_v, out_hbm.at[pl.ds(base + g * CHUNK, CHUNK)])


def kernel(token_ids, embeddings):
    flat = token_ids.reshape(NW * N_CHUNK, CHUNK)
    out = _embed_sc(embeddings, flat)
    return out.reshape(B, S, D)


# speedup vs baseline: 1.1162x; 1.1162x over previous
import functools

import jax
import jax.numpy as jnp
from jax import lax
from jax.experimental import pallas as pl
from jax.experimental.pallas import tpu as pltpu
from jax.experimental.pallas import tpu_sc as plsc

NUM_EMB = 1000000
D = 64
B = 4096
S = 200
TOTAL = B * S
NC = 2
NS = 16
NW = NC * NS
PER_W = TOTAL // NW
CHUNK = 128
N_CHUNK = PER_W // CHUNK
NBUF = 4
N_ROUND = N_CHUNK // NBUF

_mesh = plsc.VectorSubcoreMesh(core_axis_name="c", subcore_axis_name="s")


@functools.partial(
    pl.kernel,
    out_type=jax.ShapeDtypeStruct((TOTAL, D), jnp.float32),
    mesh=_mesh,
    compiler_params=pltpu.CompilerParams(use_tc_tiling_on_sc=False),
    scratch_types=[
        pltpu.VMEM((N_CHUNK, CHUNK), jnp.int32),
        pltpu.VMEM((NBUF, CHUNK, D), jnp.float32),
        pltpu.SemaphoreType.DMA((NBUF,)),
        pltpu.SemaphoreType.DMA((NBUF,)),
    ],
)
def _embed_sc(table_hbm, idx_hbm, out_hbm, idx_v, rows_v, gsem, ssem):
    wid = lax.axis_index("s") * NC + lax.axis_index("c")
    pltpu.sync_copy(idx_hbm.at[pl.ds(wid * N_CHUNK, N_CHUNK)], idx_v)
    base = wid * PER_W

    def wait_gather(b):
        pltpu.make_async_copy(
            table_hbm.at[idx_v.at[b]], rows_v.at[b], gsem.at[b]).wait()

    def wait_store(b):
        pltpu.make_async_copy(
            rows_v.at[b], out_hbm.at[pl.ds(base, CHUNK)], ssem.at[b]).wait()

    for b in range(NBUF):
        pltpu.async_copy(table_hbm.at[idx_v.at[b]], rows_v.at[b], gsem.at[b])

    @pl.loop(0, N_ROUND - 1)
    def _round(r):
        g0 = r * NBUF
        for b in range(NBUF):
            wait_gather(b)
            pltpu.async_copy(
                rows_v.at[b],
                out_hbm.at[pl.ds(base + (g0 + b) * CHUNK, CHUNK)],
                ssem.at[b])
        for b in range(NBUF):
            wait_store(b)
            pltpu.async_copy(
                table_hbm.at[idx_v.at[g0 + NBUF + b]], rows_v.at[b], gsem.at[b])

    g0 = (N_ROUND - 1) * NBUF
    for b in range(NBUF):
        wait_gather(b)
        pltpu.async_copy(
            rows_v.at[b],
            out_hbm.at[pl.ds(base + (g0 + b) * CHUNK, CHUNK)],
            ssem.at[b])
    for b in range(NBUF):
        wait_store(b)


def kernel(token_ids, embeddings):
    flat = token_ids.reshape(NW * N_CHUNK, CHUNK)
    out = _embed_sc(embeddings, flat)
    return out.reshape(B, S, D)
